# Initial kernel scaffold; baseline (speedup 1.0000x reference)
import jax
import jax.numpy as jnp
from jax.experimental import pallas as pl


def kernel(x, W, b, gamma, beta, edge_index):
    raise NotImplementedError("write your pallas kernel here")


# speedup vs baseline: 17.0762x; 17.0762x over previous
import functools

import jax
import jax.numpy as jnp
from jax import lax
from jax.experimental import pallas as pl
from jax.experimental.pallas import tpu as pltpu
from jax.experimental.pallas import tpu_sc as plsc

N = 10000
E = 320000
D = 128

NC = 2
NS = 16
NW = NC * NS
EW = E // NW
CHUNK = 80
NCHUNK = EW // CHUNK
NP = 10240
NPR = 10240
ROWS_W = NPR // NS
DEGS_W = NP // NS

@functools.cache
def _deg_sc_build():
    return pl.kernel(
        _deg_sc_body,
        out_type=jax.ShapeDtypeStruct((NC * NP,), jnp.float32),
        mesh=plsc.VectorSubcoreMesh(core_axis_name="c", subcore_axis_name="s"),
        scratch_types=[
            pltpu.VMEM((CHUNK,), jnp.int32),
            pltpu.VMEM((CHUNK,), jnp.float32),
            pltpu.VMEM_SHARED((NP,), jnp.float32),
        ],
    )


def _deg_sc(dst, zeros_np, ones_c):
    return _deg_sc_build()(dst, zeros_np, ones_c)


def _deg_sc_body(dst_hbm, zeros_hbm, ones_hbm, out_hbm, idx_v, ones_v, deg_sh):
    cid = lax.axis_index("c")
    sid = lax.axis_index("s")
    w = cid * NS + sid

    pltpu.sync_copy(zeros_hbm.at[pl.ds(sid * DEGS_W, DEGS_W)],
                    deg_sh.at[pl.ds(sid * DEGS_W, DEGS_W)])
    pltpu.sync_copy(ones_hbm, ones_v)
    plsc.subcore_barrier()

    def body(j, carry):
        base = w * EW + j * CHUNK
        pltpu.sync_copy(dst_hbm.at[pl.ds(base, CHUNK)], idx_v)
        pltpu.sync_copy(ones_v, deg_sh.at[idx_v], add=True)
        return carry

    lax.fori_loop(0, NCHUNK, body, 0)
    plsc.subcore_barrier()

    pltpu.sync_copy(deg_sh.at[pl.ds(sid * DEGS_W, DEGS_W)],
                    out_hbm.at[pl.ds(cid * NP + sid * DEGS_W, DEGS_W)])


@functools.cache
def _scatter_sc_build():
    return pl.kernel(
        _scatter_sc_body,
        out_type=jax.ShapeDtypeStruct((NC, NPR, D), jnp.float32),
        mesh=plsc.VectorSubcoreMesh(core_axis_name="c", subcore_axis_name="s"),
        scratch_types=[
            pltpu.VMEM((CHUNK,), jnp.int32),
            pltpu.VMEM((CHUNK,), jnp.int32),
            pltpu.VMEM((CHUNK, D), jnp.float32),
            pltpu.VMEM_SHARED((NPR, D), jnp.float32),
            pltpu.SemaphoreType.DMA,
        ],
    )


def _scatter_sc(src, dst, y, zeros_nd):
    return _scatter_sc_build()(src, dst, y, zeros_nd)


def _scatter_sc_body(src_hbm, dst_hbm, y_hbm, zeros_hbm, out_hbm,
                     src_v, dst_v, rows_v, acc_sh, gsem):
    cid = lax.axis_index("c")
    sid = lax.axis_index("s")
    w = cid * NS + sid

    pltpu.sync_copy(zeros_hbm.at[pl.ds(sid * ROWS_W, ROWS_W)],
                    acc_sh.at[pl.ds(sid * ROWS_W, ROWS_W)])
    plsc.subcore_barrier()

    def body(j, carry):
        base = w * EW + j * CHUNK
        pltpu.sync_copy(src_hbm.at[pl.ds(base, CHUNK)], src_v)
        pltpu.sync_copy(dst_hbm.at[pl.ds(base, CHUNK)], dst_v)
        pltpu.async_copy(y_hbm.at[src_v], rows_v, gsem).wait()
        pltpu.sync_copy(rows_v, acc_sh.at[dst_v], add=True)
        return carry

    lax.fori_loop(0, NCHUNK, body, 0)
    plsc.subcore_barrier()

    pltpu.sync_copy(acc_sh.at[pl.ds(sid * ROWS_W, ROWS_W)],
                    out_hbm.at[cid, pl.ds(sid * ROWS_W, ROWS_W)])


BR = 2000
NBLK = N // BR


def _mm_body(x_ref, w_ref, o_ref):
    o_ref[...] = jnp.dot(x_ref[...], w_ref[...],
                         preferred_element_type=jnp.float32)


def _matmul_tc(x, W):
    return pl.pallas_call(
        _mm_body,
        grid=(NBLK,),
        in_specs=[
            pl.BlockSpec((BR, D), lambda i: (i, 0)),
            pl.BlockSpec((D, D), lambda i: (0, 0)),
        ],
        out_specs=pl.BlockSpec((BR, D), lambda i: (i, 0)),
        out_shape=jax.ShapeDtypeStruct((N, D), jnp.float32),
    )(x, W)


def _dinv(dt_ref):
    return lax.rsqrt(1.0 + dt_ref[:, 0] + dt_ref[:, 1])


def _scale_body(x_ref, dt_ref, o_ref):
    o_ref[...] = x_ref[...] * _dinv(dt_ref)[:, None]


def _scale_tc(x_lin, deg_t):
    return pl.pallas_call(
        _scale_body,
        grid=(NBLK,),
        in_specs=[
            pl.BlockSpec((BR, D), lambda i: (i, 0)),
            pl.BlockSpec((BR, 2), lambda i: (i, 0)),
        ],
        out_specs=pl.BlockSpec((BR, D), lambda i: (i, 0)),
        out_shape=jax.ShapeDtypeStruct((N, D), jnp.float32),
    )(x_lin, deg_t)


def _stats_body(a0_ref, a1_ref, y_ref, dt_ref, sum_ref, sq_ref):
    i = pl.program_id(0)
    z = (a0_ref[...] + a1_ref[...] + y_ref[...]) * _dinv(dt_ref)[:, None]

    @pl.when(i == 0)
    def _():
        sum_ref[...] = jnp.zeros_like(sum_ref)
        sq_ref[...] = jnp.zeros_like(sq_ref)

    sum_ref[...] += jnp.sum(z, axis=0, keepdims=True)
    sq_ref[...] += jnp.sum(z * z, axis=0, keepdims=True)


def _stats_tc(a0, a1, y, deg_t):
    return pl.pallas_call(
        _stats_body,
        grid=(NBLK,),
        in_specs=[
            pl.BlockSpec((BR, D), lambda i: (i, 0)),
            pl.BlockSpec((BR, D), lambda i: (i, 0)),
            pl.BlockSpec((BR, D), lambda i: (i, 0)),
            pl.BlockSpec((BR, 2), lambda i: (i, 0)),
        ],
        out_specs=[
            pl.BlockSpec((1, D), lambda i: (0, 0)),
            pl.BlockSpec((1, D), lambda i: (0, 0)),
        ],
        out_shape=[
            jax.ShapeDtypeStruct((1, D), jnp.float32),
            jax.ShapeDtypeStruct((1, D), jnp.float32),
        ],
    )(a0, a1, y, deg_t)


def _final_body(a0_ref, a1_ref, y_ref, dt_ref, sum_ref, sq_ref,
                g_ref, b_ref, o_ref):
    z = (a0_ref[...] + a1_ref[...] + y_ref[...]) * _dinv(dt_ref)[:, None]
    mean = sum_ref[...] * (1.0 / N)
    var = sq_ref[...] * (1.0 / N) - mean * mean
    scale = lax.rsqrt(var + 1e-5) * g_ref[...]
    o_ref[...] = jnp.maximum((z - mean) * scale + b_ref[...], 0.0)


def _final_tc(a0, a1, y, deg_t, s, sq, gamma, beta):
    return pl.pallas_call(
        _final_body,
        grid=(NBLK,),
        in_specs=[
            pl.BlockSpec((BR, D), lambda i: (i, 0)),
            pl.BlockSpec((BR, D), lambda i: (i, 0)),
            pl.BlockSpec((BR, D), lambda i: (i, 0)),
            pl.BlockSpec((BR, 2), lambda i: (i, 0)),
            pl.BlockSpec((1, D), lambda i: (0, 0)),
            pl.BlockSpec((1, D), lambda i: (0, 0)),
            pl.BlockSpec((1, D), lambda i: (0, 0)),
            pl.BlockSpec((1, D), lambda i: (0, 0)),
        ],
        out_specs=pl.BlockSpec((BR, D), lambda i: (i, 0)),
        out_shape=jax.ShapeDtypeStruct((N, D), jnp.float32),
    )(a0, a1, y, deg_t, s, sq, gamma, beta)


def kernel(x, W, b, gamma, beta, edge_index):
    del b
    ei = edge_index.astype(jnp.int32)
    src = ei[0]
    dst = ei[1]

    zeros_np = jnp.zeros((NP,), jnp.float32)
    ones_c = jnp.ones((CHUNK,), jnp.float32)
    zeros_nd = jnp.zeros((NPR, D), jnp.float32)

    degf = _deg_sc(dst, zeros_np, ones_c)
    x_lin = _matmul_tc(x, W)
    deg_t = degf.reshape(NC, NP).T[:N]
    y = _scale_tc(x_lin, deg_t)
    acc = _scatter_sc(src, dst, y, zeros_nd)
    a0, a1 = acc[0, :N], acc[1, :N]
    s, sq = _stats_tc(a0, a1, y, deg_t)
    out = _final_tc(a0, a1, y, deg_t, s, sq,
                    gamma.reshape(1, D), beta.reshape(1, D))
    return out


# speedup vs baseline: 30.8015x; 1.8038x over previous
import functools

import jax
import jax.numpy as jnp
from jax import lax
from jax.experimental import pallas as pl
from jax.experimental.pallas import tpu as pltpu
from jax.experimental.pallas import tpu_sc as plsc

N = 10000
E = 320000
D = 128

NC = 2
NS = 16
NW = NC * NS
EW = E // NW
CHUNK = 80
NCHUNK = EW // CHUNK
NP = 10240
NPR = 10240
ROWS_W = NPR // NS
DEGS_W = NP // NS
NG = 5
CPG = NCHUNK // NG

@functools.cache
def _deg_sc_build():
    return pl.kernel(
        _deg_sc_body,
        out_type=jax.ShapeDtypeStruct((NC * NP,), jnp.float32),
        mesh=plsc.VectorSubcoreMesh(core_axis_name="c", subcore_axis_name="s"),
        scratch_types=[
            pltpu.VMEM((CPG, CHUNK), jnp.int32),
            pltpu.VMEM((CHUNK,), jnp.float32),
            pltpu.VMEM_SHARED((NP,), jnp.float32),
            pltpu.SemaphoreType.DMA,
        ],
    )


def _deg_sc(dst3, zeros_np, ones_c):
    return _deg_sc_build()(dst3, zeros_np, ones_c)


def _deg_sc_body(dst_hbm, zeros_hbm, ones_hbm, out_hbm, idx_v, ones_v,
                 deg_sh, ssem):
    cid = lax.axis_index("c")
    sid = lax.axis_index("s")
    w = cid * NS + sid

    pltpu.sync_copy(zeros_hbm.at[pl.ds(sid * DEGS_W, DEGS_W)],
                    deg_sh.at[pl.ds(sid * DEGS_W, DEGS_W)])
    pltpu.sync_copy(ones_hbm, ones_v)
    plsc.subcore_barrier()

    def group(g, carry):
        pltpu.sync_copy(dst_hbm.at[w, g], idx_v)

        def body(j, c2):
            pltpu.async_copy(ones_v, deg_sh.at[idx_v.at[j]], ssem, add=True)
            return c2

        lax.fori_loop(0, CPG, body, 0)

        def drain(j, c2):
            pltpu.make_async_copy(ones_v, deg_sh.at[idx_v.at[0]], ssem).wait()
            return c2

        lax.fori_loop(0, CPG, drain, 0)
        return carry

    lax.fori_loop(0, NG, group, 0)
    plsc.subcore_barrier()

    pltpu.sync_copy(deg_sh.at[pl.ds(sid * DEGS_W, DEGS_W)],
                    out_hbm.at[pl.ds(cid * NP + sid * DEGS_W, DEGS_W)])


@functools.cache
def _scatter_sc_build():
    return pl.kernel(
        _scatter_sc_body,
        out_type=jax.ShapeDtypeStruct((NC, NPR, D), jnp.float32),
        mesh=plsc.VectorSubcoreMesh(core_axis_name="c", subcore_axis_name="s"),
        scratch_types=[
            pltpu.VMEM((CPG, CHUNK), jnp.int32),
            pltpu.VMEM((CPG, CHUNK), jnp.int32),
            pltpu.VMEM((CHUNK, D), jnp.float32),
            pltpu.VMEM((CHUNK, D), jnp.float32),
            pltpu.VMEM_SHARED((NPR, D), jnp.float32),
            pltpu.SemaphoreType.DMA,
            pltpu.SemaphoreType.DMA,
            pltpu.SemaphoreType.DMA,
            pltpu.SemaphoreType.DMA,
        ],
    )


def _scatter_sc(src3, dst3, y, zeros_nd):
    return _scatter_sc_build()(src3, dst3, y, zeros_nd)


def _scatter_sc_body(src_hbm, dst_hbm, y_hbm, zeros_hbm, out_hbm,
                     src_v, dst_v, rows0, rows1, acc_sh,
                     gsem0, gsem1, ssem0, ssem1):
    cid = lax.axis_index("c")
    sid = lax.axis_index("s")
    w = cid * NS + sid
    rows = (rows0, rows1)
    gsem = (gsem0, gsem1)
    ssem = (ssem0, ssem1)

    pltpu.sync_copy(zeros_hbm.at[pl.ds(sid * ROWS_W, ROWS_W)],
                    acc_sh.at[pl.ds(sid * ROWS_W, ROWS_W)])
    plsc.subcore_barrier()

    def gather(j, b):
        return pltpu.async_copy(y_hbm.at[src_v.at[j]], rows[b], gsem[b])

    def scatter(j, b):
        return pltpu.async_copy(rows[b], acc_sh.at[dst_v.at[j]], ssem[b],
                                add=True)

    def wait_g(b):
        pltpu.make_async_copy(y_hbm.at[src_v.at[0]], rows[b], gsem[b]).wait()

    def wait_s(b):
        pltpu.make_async_copy(rows[b], acc_sh.at[dst_v.at[0]], ssem[b]).wait()

    def group(g, carry):
        pltpu.sync_copy(src_hbm.at[w, g], src_v)
        pltpu.sync_copy(dst_hbm.at[w, g], dst_v)
        gather(0, 0)
        wait_g(0)
        scatter(0, 0)
        gather(1, 1)
        wait_g(1)
        scatter(1, 1)
        wait_s(0)
        gather(2, 0)

        def body(i, c2):
            j = 2 * i
            wait_g(0)
            scatter(j, 0)
            wait_s(1)
            gather(j + 1, 1)
            wait_g(1)
            scatter(j + 1, 1)
            wait_s(0)
            gather(j + 2, 0)
            return c2

        lax.fori_loop(1, (CPG - 1) // 2, body, 0)
        wait_g(0)
        scatter(CPG - 1, 0)
        wait_s(1)
        wait_s(0)
        return carry

    lax.fori_loop(0, NG, group, 0)
    plsc.subcore_barrier()

    pltpu.sync_copy(acc_sh.at[pl.ds(sid * ROWS_W, ROWS_W)],
                    out_hbm.at[cid, pl.ds(sid * ROWS_W, ROWS_W)])


BR = 2000
NBLK = N // BR


def _mm_body(x_ref, w_ref, o_ref):
    o_ref[...] = jnp.dot(x_ref[...], w_ref[...],
                         preferred_element_type=jnp.float32)


def _matmul_tc(x, W):
    return pl.pallas_call(
        _mm_body,
        grid=(NBLK,),
        in_specs=[
            pl.BlockSpec((BR, D), lambda i: (i, 0)),
            pl.BlockSpec((D, D), lambda i: (0, 0)),
        ],
        out_specs=pl.BlockSpec((BR, D), lambda i: (i, 0)),
        out_shape=jax.ShapeDtypeStruct((N, D), jnp.float32),
    )(x, W)


def _dinv(dt_ref):
    return lax.rsqrt(1.0 + dt_ref[:, 0] + dt_ref[:, 1])


def _scale_body(x_ref, dt_ref, o_ref):
    o_ref[...] = x_ref[...] * _dinv(dt_ref)[:, None]


def _scale_tc(x_lin, deg_t):
    return pl.pallas_call(
        _scale_body,
        grid=(NBLK,),
        in_specs=[
            pl.BlockSpec((BR, D), lambda i: (i, 0)),
            pl.BlockSpec((BR, 2), lambda i: (i, 0)),
        ],
        out_specs=pl.BlockSpec((BR, D), lambda i: (i, 0)),
        out_shape=jax.ShapeDtypeStruct((N, D), jnp.float32),
    )(x_lin, deg_t)


def _stats_body(a0_ref, a1_ref, y_ref, dt_ref, sum_ref, sq_ref):
    i = pl.program_id(0)
    z = (a0_ref[...] + a1_ref[...] + y_ref[...]) * _dinv(dt_ref)[:, None]

    @pl.when(i == 0)
    def _():
        sum_ref[...] = jnp.zeros_like(sum_ref)
        sq_ref[...] = jnp.zeros_like(sq_ref)

    sum_ref[...] += jnp.sum(z, axis=0, keepdims=True)
    sq_ref[...] += jnp.sum(z * z, axis=0, keepdims=True)


def _stats_tc(a0, a1, y, deg_t):
    return pl.pallas_call(
        _stats_body,
        grid=(NBLK,),
        in_specs=[
            pl.BlockSpec((BR, D), lambda i: (i, 0)),
            pl.BlockSpec((BR, D), lambda i: (i, 0)),
            pl.BlockSpec((BR, D), lambda i: (i, 0)),
            pl.BlockSpec((BR, 2), lambda i: (i, 0)),
        ],
        out_specs=[
            pl.BlockSpec((1, D), lambda i: (0, 0)),
            pl.BlockSpec((1, D), lambda i: (0, 0)),
        ],
        out_shape=[
            jax.ShapeDtypeStruct((1, D), jnp.float32),
            jax.ShapeDtypeStruct((1, D), jnp.float32),
        ],
    )(a0, a1, y, deg_t)


def _final_body(a0_ref, a1_ref, y_ref, dt_ref, sum_ref, sq_ref,
                g_ref, b_ref, o_ref):
    z = (a0_ref[...] + a1_ref[...] + y_ref[...]) * _dinv(dt_ref)[:, None]
    mean = sum_ref[...] * (1.0 / N)
    var = sq_ref[...] * (1.0 / N) - mean * mean
    scale = lax.rsqrt(var + 1e-5) * g_ref[...]
    o_ref[...] = jnp.maximum((z - mean) * scale + b_ref[...], 0.0)


def _final_tc(a0, a1, y, deg_t, s, sq, gamma, beta):
    return pl.pallas_call(
        _final_body,
        grid=(NBLK,),
        in_specs=[
            pl.BlockSpec((BR, D), lambda i: (i, 0)),
            pl.BlockSpec((BR, D), lambda i: (i, 0)),
            pl.BlockSpec((BR, D), lambda i: (i, 0)),
            pl.BlockSpec((BR, 2), lambda i: (i, 0)),
            pl.BlockSpec((1, D), lambda i: (0, 0)),
            pl.BlockSpec((1, D), lambda i: (0, 0)),
            pl.BlockSpec((1, D), lambda i: (0, 0)),
            pl.BlockSpec((1, D), lambda i: (0, 0)),
        ],
        out_specs=pl.BlockSpec((BR, D), lambda i: (i, 0)),
        out_shape=jax.ShapeDtypeStruct((N, D), jnp.float32),
    )(a0, a1, y, deg_t, s, sq, gamma, beta)


def kernel(x, W, b, gamma, beta, edge_index):
    del b
    ei = edge_index.astype(jnp.int32)
    src3 = ei[0].reshape(NW, NG, CPG, CHUNK)
    dst3 = ei[1].reshape(NW, NG, CPG, CHUNK)

    zeros_np = jnp.zeros((NP,), jnp.float32)
    ones_c = jnp.ones((CHUNK,), jnp.float32)
    zeros_nd = jnp.zeros((NPR, D), jnp.float32)

    degf = _deg_sc(dst3, zeros_np, ones_c)
    x_lin = _matmul_tc(x, W)
    deg_t = degf.reshape(NC, NP).T[:N]
    y = _scale_tc(x_lin, deg_t)
    acc = _scatter_sc(src3, dst3, y, zeros_nd)
    a0, a1 = acc[0, :N], acc[1, :N]
    s, sq = _stats_tc(a0, a1, y, deg_t)
    out = _final_tc(a0, a1, y, deg_t, s, sq,
                    gamma.reshape(1, D), beta.reshape(1, D))
    return out


# speedup vs baseline: 38.1951x; 1.2400x over previous
import functools

import jax
import jax.numpy as jnp
from jax import lax
from jax.experimental import pallas as pl
from jax.experimental.pallas import tpu as pltpu
from jax.experimental.pallas import tpu_sc as plsc

N = 10000
E = 320000
D = 128

NC = 2
NS = 16
NW = NC * NS
EW = E // NW
CHUNK = 80
NCHUNK = EW // CHUNK
NP = 10240
NPR = 10112
ROWS_W = NPR // NS
DEGS_W = NP // NS
NG = 5
CPG = NCHUNK // NG

@functools.cache
def _deg_sc_build():
    return pl.kernel(
        _deg_sc_body,
        out_type=jax.ShapeDtypeStruct((NC * NP,), jnp.float32),
        mesh=plsc.VectorSubcoreMesh(core_axis_name="c", subcore_axis_name="s"),
        scratch_types=[
            pltpu.VMEM((CPG, CHUNK), jnp.int32),
            pltpu.VMEM((CHUNK,), jnp.float32),
            pltpu.VMEM_SHARED((NP,), jnp.float32),
            pltpu.SemaphoreType.DMA,
        ],
    )


def _deg_sc(dst3, zeros_np, ones_c):
    return _deg_sc_build()(dst3, zeros_np, ones_c)


def _deg_sc_body(dst_hbm, zeros_hbm, ones_hbm, out_hbm, idx_v, ones_v,
                 deg_sh, ssem):
    cid = lax.axis_index("c")
    sid = lax.axis_index("s")
    w = cid * NS + sid

    pltpu.sync_copy(zeros_hbm.at[pl.ds(sid * DEGS_W, DEGS_W)],
                    deg_sh.at[pl.ds(sid * DEGS_W, DEGS_W)])
    pltpu.sync_copy(ones_hbm, ones_v)
    plsc.subcore_barrier()

    def group(g, carry):
        pltpu.sync_copy(dst_hbm.at[w, g], idx_v)

        def body(j, c2):
            pltpu.async_copy(ones_v, deg_sh.at[idx_v.at[j]], ssem, add=True)
            return c2

        lax.fori_loop(0, CPG, body, 0)

        def drain(j, c2):
            pltpu.make_async_copy(ones_v, deg_sh.at[idx_v.at[0]], ssem).wait()
            return c2

        lax.fori_loop(0, CPG, drain, 0)
        return carry

    lax.fori_loop(0, NG, group, 0)
    plsc.subcore_barrier()

    pltpu.sync_copy(deg_sh.at[pl.ds(sid * DEGS_W, DEGS_W)],
                    out_hbm.at[pl.ds(cid * NP + sid * DEGS_W, DEGS_W)])


@functools.cache
def _scatter_sc_build():
    return pl.kernel(
        _scatter_sc_body,
        out_type=jax.ShapeDtypeStruct((NC, NPR, D), jnp.float32),
        mesh=plsc.VectorSubcoreMesh(core_axis_name="c", subcore_axis_name="s"),
        scratch_types=[
            pltpu.VMEM((CPG, CHUNK), jnp.int32),
            pltpu.VMEM((CPG, CHUNK), jnp.int32),
            pltpu.VMEM((CHUNK, D), jnp.float32),
            pltpu.VMEM((CHUNK, D), jnp.float32),
            pltpu.VMEM((CHUNK, D), jnp.float32),
            pltpu.VMEM((CHUNK, D), jnp.float32),
            pltpu.VMEM_SHARED((NPR, D), jnp.float32),
            pltpu.SemaphoreType.DMA,
            pltpu.SemaphoreType.DMA,
            pltpu.SemaphoreType.DMA,
            pltpu.SemaphoreType.DMA,
            pltpu.SemaphoreType.DMA,
            pltpu.SemaphoreType.DMA,
            pltpu.SemaphoreType.DMA,
            pltpu.SemaphoreType.DMA,
        ],
    )


def _scatter_sc(src3, dst3, y, zeros_nd):
    return _scatter_sc_build()(src3, dst3, y, zeros_nd)


def _scatter_sc_body(src_hbm, dst_hbm, y_hbm, zeros_hbm, out_hbm,
                     src_v, dst_v, rows0, rows1, rows2, rows3, acc_sh,
                     gsem0, gsem1, gsem2, gsem3,
                     ssem0, ssem1, ssem2, ssem3):
    cid = lax.axis_index("c")
    sid = lax.axis_index("s")
    w = cid * NS + sid
    rows = (rows0, rows1, rows2, rows3)
    gsem = (gsem0, gsem1, gsem2, gsem3)
    ssem = (ssem0, ssem1, ssem2, ssem3)

    pltpu.sync_copy(zeros_hbm.at[pl.ds(sid * ROWS_W, ROWS_W)],
                    acc_sh.at[pl.ds(sid * ROWS_W, ROWS_W)])
    plsc.subcore_barrier()

    def gather(j, b):
        return pltpu.async_copy(y_hbm.at[src_v.at[j]], rows[b], gsem[b])

    def scatter(j, b):
        return pltpu.async_copy(rows[b], acc_sh.at[dst_v.at[j]], ssem[b],
                                add=True)

    def wait_g(b):
        pltpu.make_async_copy(y_hbm.at[src_v.at[0]], rows[b], gsem[b]).wait()

    def wait_s(b):
        pltpu.make_async_copy(rows[b], acc_sh.at[dst_v.at[0]], ssem[b]).wait()

    def group(g, carry):
        pltpu.sync_copy(src_hbm.at[w, g], src_v)
        pltpu.sync_copy(dst_hbm.at[w, g], dst_v)
        gather(0, 0)
        gather(1, 1)
        gather(2, 2)
        wait_g(0); scatter(0, 0); gather(3, 3)
        wait_g(1); scatter(1, 1); wait_s(0); gather(4, 0)
        wait_g(2); scatter(2, 2); wait_s(1); gather(5, 1)
        wait_g(3); scatter(3, 3); wait_s(2); gather(6, 2)

        def body(i, c2):
            for p in range(4):
                j = 4 * i + p
                bn = (p + 3) % 4
                wait_g(p)
                scatter(j, p)
                wait_s(bn)
                gather(j + 3, bn)
            return c2

        lax.fori_loop(1, 5, body, 0)
        wait_g(0); scatter(20, 0); wait_s(3); gather(23, 3)
        wait_g(1); scatter(21, 1); wait_s(0); gather(24, 0)
        wait_g(2); scatter(22, 2); wait_s(1)
        wait_g(3); scatter(23, 3); wait_s(2)
        wait_g(0); scatter(24, 0); wait_s(3); wait_s(0)
        return carry

    lax.fori_loop(0, NG, group, 0)
    plsc.subcore_barrier()

    pltpu.sync_copy(acc_sh.at[pl.ds(sid * ROWS_W, ROWS_W)],
                    out_hbm.at[cid, pl.ds(sid * ROWS_W, ROWS_W)])


BR = 2000
NBLK = N // BR


def _mm_body(x_ref, w_ref, o_ref):
    o_ref[...] = jnp.dot(x_ref[...], w_ref[...],
                         preferred_element_type=jnp.float32)


def _matmul_tc(x, W):
    return pl.pallas_call(
        _mm_body,
        grid=(NBLK,),
        in_specs=[
            pl.BlockSpec((BR, D), lambda i: (i, 0)),
            pl.BlockSpec((D, D), lambda i: (0, 0)),
        ],
        out_specs=pl.BlockSpec((BR, D), lambda i: (i, 0)),
        out_shape=jax.ShapeDtypeStruct((N, D), jnp.float32),
    )(x, W)


def _dinv(dt_ref):
    return lax.rsqrt(1.0 + dt_ref[:, 0] + dt_ref[:, 1])


def _scale_body(x_ref, dt_ref, o_ref):
    o_ref[...] = x_ref[...] * _dinv(dt_ref)[:, None]


def _scale_tc(x_lin, deg_t):
    return pl.pallas_call(
        _scale_body,
        grid=(NBLK,),
        in_specs=[
            pl.BlockSpec((BR, D), lambda i: (i, 0)),
            pl.BlockSpec((BR, 2), lambda i: (i, 0)),
        ],
        out_specs=pl.BlockSpec((BR, D), lambda i: (i, 0)),
        out_shape=jax.ShapeDtypeStruct((N, D), jnp.float32),
    )(x_lin, deg_t)


def _stats_body(a0_ref, a1_ref, y_ref, dt_ref, sum_ref, sq_ref):
    i = pl.program_id(0)
    z = (a0_ref[...] + a1_ref[...] + y_ref[...]) * _dinv(dt_ref)[:, None]

    @pl.when(i == 0)
    def _():
        sum_ref[...] = jnp.zeros_like(sum_ref)
        sq_ref[...] = jnp.zeros_like(sq_ref)

    sum_ref[...] += jnp.sum(z, axis=0, keepdims=True)
    sq_ref[...] += jnp.sum(z * z, axis=0, keepdims=True)


def _stats_tc(a0, a1, y, deg_t):
    return pl.pallas_call(
        _stats_body,
        grid=(NBLK,),
        in_specs=[
            pl.BlockSpec((BR, D), lambda i: (i, 0)),
            pl.BlockSpec((BR, D), lambda i: (i, 0)),
            pl.BlockSpec((BR, D), lambda i: (i, 0)),
            pl.BlockSpec((BR, 2), lambda i: (i, 0)),
        ],
        out_specs=[
            pl.BlockSpec((1, D), lambda i: (0, 0)),
            pl.BlockSpec((1, D), lambda i: (0, 0)),
        ],
        out_shape=[
            jax.ShapeDtypeStruct((1, D), jnp.float32),
            jax.ShapeDtypeStruct((1, D), jnp.float32),
        ],
    )(a0, a1, y, deg_t)


def _final_body(a0_ref, a1_ref, y_ref, dt_ref, sum_ref, sq_ref,
                g_ref, b_ref, o_ref):
    z = (a0_ref[...] + a1_ref[...] + y_ref[...]) * _dinv(dt_ref)[:, None]
    mean = sum_ref[...] * (1.0 / N)
    var = sq_ref[...] * (1.0 / N) - mean * mean
    scale = lax.rsqrt(var + 1e-5) * g_ref[...]
    o_ref[...] = jnp.maximum((z - mean) * scale + b_ref[...], 0.0)


def _final_tc(a0, a1, y, deg_t, s, sq, gamma, beta):
    return pl.pallas_call(
        _final_body,
        grid=(NBLK,),
        in_specs=[
            pl.BlockSpec((BR, D), lambda i: (i, 0)),
            pl.BlockSpec((BR, D), lambda i: (i, 0)),
            pl.BlockSpec((BR, D), lambda i: (i, 0)),
            pl.BlockSpec((BR, 2), lambda i: (i, 0)),
            pl.BlockSpec((1, D), lambda i: (0, 0)),
            pl.BlockSpec((1, D), lambda i: (0, 0)),
            pl.BlockSpec((1, D), lambda i: (0, 0)),
            pl.BlockSpec((1, D), lambda i: (0, 0)),
        ],
        out_specs=pl.BlockSpec((BR, D), lambda i: (i, 0)),
        out_shape=jax.ShapeDtypeStruct((N, D), jnp.float32),
    )(a0, a1, y, deg_t, s, sq, gamma, beta)


def kernel(x, W, b, gamma, beta, edge_index):
    del b
    ei = edge_index.astype(jnp.int32)
    src3 = ei[0].reshape(NW, NG, CPG, CHUNK)
    dst3 = ei[1].reshape(NW, NG, CPG, CHUNK)

    zeros_np = jnp.zeros((NP,), jnp.float32)
    ones_c = jnp.ones((CHUNK,), jnp.float32)
    zeros_nd = jnp.zeros((NPR, D), jnp.float32)

    degf = _deg_sc(dst3, zeros_np, ones_c)
    x_lin = _matmul_tc(x, W)
    deg_t = degf.reshape(NC, NP).T[:N]
    y = _scale_tc(x_lin, deg_t)
    acc = _scatter_sc(src3, dst3, y, zeros_nd)
    a0, a1 = acc[0, :N], acc[1, :N]
    s, sq = _stats_tc(a0, a1, y, deg_t)
    out = _final_tc(a0, a1, y, deg_t, s, sq,
                    gamma.reshape(1, D), beta.reshape(1, D))
    return out


# speedup vs baseline: 38.5425x; 1.0091x over previous
import functools

import jax
import jax.numpy as jnp
from jax import lax
from jax.experimental import pallas as pl
from jax.experimental.pallas import tpu as pltpu
from jax.experimental.pallas import tpu_sc as plsc

N = 10000
E = 320000
D = 128

NC = 2
NS = 16
NW = NC * NS
EW = E // NW
CHUNK = 80
NCHUNK = EW // CHUNK
NP = 10240
NPR = 10112
ROWS_W = NPR // NS
DEGS_W = NP // NS
NG = 5
CPG = NCHUNK // NG

@functools.cache
def _deg_sc_build():
    return pl.kernel(
        _deg_sc_body,
        out_type=jax.ShapeDtypeStruct((NC * NP,), jnp.float32),
        mesh=plsc.VectorSubcoreMesh(core_axis_name="c", subcore_axis_name="s"),
        scratch_types=[
            pltpu.VMEM((CPG, CHUNK), jnp.int32),
            pltpu.VMEM((CHUNK,), jnp.float32),
            pltpu.VMEM_SHARED((NP,), jnp.float32),
            pltpu.SemaphoreType.DMA,
        ],
    )


def _deg_sc(dst3, zeros_np, ones_c):
    return _deg_sc_build()(dst3, zeros_np, ones_c)


def _deg_sc_body(dst_hbm, zeros_hbm, ones_hbm, out_hbm, idx_v, ones_v,
                 deg_sh, ssem):
    cid = lax.axis_index("c")
    sid = lax.axis_index("s")
    w = cid * NS + sid

    pltpu.sync_copy(zeros_hbm.at[pl.ds(sid * DEGS_W, DEGS_W)],
                    deg_sh.at[pl.ds(sid * DEGS_W, DEGS_W)])
    pltpu.sync_copy(ones_hbm, ones_v)
    plsc.subcore_barrier()

    def group(g, carry):
        pltpu.sync_copy(dst_hbm.at[w, g], idx_v)

        def body(j, c2):
            pltpu.async_copy(ones_v, deg_sh.at[idx_v.at[j]], ssem, add=True)
            return c2

        lax.fori_loop(0, CPG, body, 0)

        def drain(j, c2):
            pltpu.make_async_copy(ones_v, deg_sh.at[idx_v.at[0]], ssem).wait()
            return c2

        lax.fori_loop(0, CPG, drain, 0)
        return carry

    lax.fori_loop(0, NG, group, 0)
    plsc.subcore_barrier()

    pltpu.sync_copy(deg_sh.at[pl.ds(sid * DEGS_W, DEGS_W)],
                    out_hbm.at[pl.ds(cid * NP + sid * DEGS_W, DEGS_W)])


@functools.cache
def _scatter_sc_build():
    return pl.kernel(
        _scatter_sc_body,
        out_type=jax.ShapeDtypeStruct((NC, NPR, D), jnp.float32),
        mesh=plsc.VectorSubcoreMesh(core_axis_name="c", subcore_axis_name="s"),
        scratch_types=[
            pltpu.VMEM((CPG, CHUNK), jnp.int32),
            pltpu.VMEM((CPG, CHUNK), jnp.int32),
            pltpu.VMEM((CHUNK, D), jnp.float32),
            pltpu.VMEM((CHUNK, D), jnp.float32),
            pltpu.VMEM((CHUNK, D), jnp.float32),
            pltpu.VMEM((CHUNK, D), jnp.float32),
            pltpu.VMEM_SHARED((NPR, D), jnp.float32),
            pltpu.SemaphoreType.DMA,
            pltpu.SemaphoreType.DMA,
            pltpu.SemaphoreType.DMA,
            pltpu.SemaphoreType.DMA,
            pltpu.SemaphoreType.DMA,
            pltpu.SemaphoreType.DMA,
            pltpu.SemaphoreType.DMA,
            pltpu.SemaphoreType.DMA,
        ],
    )


def _scatter_sc(src3, dst3, y, zeros_nd):
    return _scatter_sc_build()(src3, dst3, y, zeros_nd)


def _scatter_sc_body(src_hbm, dst_hbm, y_hbm, zeros_hbm, out_hbm,
                     src_v, dst_v, rows0, rows1, rows2, rows3, acc_sh,
                     gsem0, gsem1, gsem2, gsem3,
                     ssem0, ssem1, ssem2, ssem3):
    cid = lax.axis_index("c")
    sid = lax.axis_index("s")
    w = cid * NS + sid
    rows = (rows0, rows1, rows2, rows3)
    gsem = (gsem0, gsem1, gsem2, gsem3)
    ssem = (ssem0, ssem1, ssem2, ssem3)

    pltpu.sync_copy(zeros_hbm.at[pl.ds(sid * ROWS_W, ROWS_W)],
                    acc_sh.at[pl.ds(sid * ROWS_W, ROWS_W)])
    plsc.subcore_barrier()

    def gather(j, b):
        return pltpu.async_copy(y_hbm.at[src_v.at[j]], rows[b], gsem[b])

    def scatter(j, b):
        return pltpu.async_copy(rows[b], acc_sh.at[dst_v.at[j]], ssem[b],
                                add=True)

    def wait_g(b):
        pltpu.make_async_copy(y_hbm.at[src_v.at[0]], rows[b], gsem[b]).wait()

    def wait_s(b):
        pltpu.make_async_copy(rows[b], acc_sh.at[dst_v.at[0]], ssem[b]).wait()

    def group(g, carry):
        pltpu.sync_copy(src_hbm.at[w, g], src_v)
        pltpu.sync_copy(dst_hbm.at[w, g], dst_v)
        gather(0, 0)
        gather(1, 1)
        gather(2, 2)
        wait_g(0); scatter(0, 0); gather(3, 3)
        wait_g(1); scatter(1, 1); wait_s(0); gather(4, 0)
        wait_g(2); scatter(2, 2); wait_s(1); gather(5, 1)
        wait_g(3); scatter(3, 3); wait_s(2); gather(6, 2)

        def body(i, c2):
            for p in range(4):
                j = 4 * i + p
                bn = (p + 3) % 4
                wait_g(p)
                scatter(j, p)
                wait_s(bn)
                gather(j + 3, bn)
            return c2

        lax.fori_loop(1, 5, body, 0)
        wait_g(0); scatter(20, 0); wait_s(3); gather(23, 3)
        wait_g(1); scatter(21, 1); wait_s(0); gather(24, 0)
        wait_g(2); scatter(22, 2); wait_s(1)
        wait_g(3); scatter(23, 3); wait_s(2)
        wait_g(0); scatter(24, 0); wait_s(3); wait_s(0)
        return carry

    lax.fori_loop(0, NG, group, 0)
    plsc.subcore_barrier()

    pltpu.sync_copy(acc_sh.at[pl.ds(sid * ROWS_W, ROWS_W)],
                    out_hbm.at[cid, pl.ds(sid * ROWS_W, ROWS_W)])


BR = 2000
NBLK = N // BR


def _dinv(dt_ref):
    return lax.rsqrt(1.0 + dt_ref[:, 0] + dt_ref[:, 1])


def _mm_body(x_ref, w_ref, dt_ref, o_ref):
    xs = x_ref[...] * _dinv(dt_ref)[:, None]
    o_ref[...] = jnp.dot(xs, w_ref[...], preferred_element_type=jnp.float32)


def _matmul_tc(x, W, deg_t):
    return pl.pallas_call(
        _mm_body,
        grid=(NBLK,),
        in_specs=[
            pl.BlockSpec((BR, D), lambda i: (i, 0)),
            pl.BlockSpec((D, D), lambda i: (0, 0)),
            pl.BlockSpec((BR, 2), lambda i: (i, 0)),
        ],
        out_specs=pl.BlockSpec((BR, D), lambda i: (i, 0)),
        out_shape=jax.ShapeDtypeStruct((N, D), jnp.float32),
    )(x, W, deg_t)


def _bn_body(a0_ref, a1_ref, y_ref, dt_ref, g_ref, b_ref, o_ref, st_ref):
    p = pl.program_id(0)
    i = pl.program_id(1)
    z = (a0_ref[...] + a1_ref[...] + y_ref[...]) * _dinv(dt_ref)[:, None]

    @pl.when(p == 0)
    def _():
        @pl.when(i == 0)
        def _():
            st_ref[...] = jnp.zeros_like(st_ref)

        st_ref[0:1, :] += jnp.sum(z, axis=0, keepdims=True)
        st_ref[1:2, :] += jnp.sum(z * z, axis=0, keepdims=True)

        @pl.when(i == NBLK - 1)
        def _():
            mean = st_ref[0:1, :] * (1.0 / N)
            var = st_ref[1:2, :] * (1.0 / N) - mean * mean
            scale = lax.rsqrt(var + 1e-5) * g_ref[...]
            st_ref[2:3, :] = scale
            st_ref[3:4, :] = b_ref[...] - mean * scale

    @pl.when(p == 1)
    def _():
        o_ref[...] = jnp.maximum(z * st_ref[2:3, :] + st_ref[3:4, :], 0.0)


def _bn_tc(a0, a1, y, deg_t, gamma, beta):
    return pl.pallas_call(
        _bn_body,
        grid=(2, NBLK),
        in_specs=[
            pl.BlockSpec((BR, D), lambda p, i: (i, 0)),
            pl.BlockSpec((BR, D), lambda p, i: (i, 0)),
            pl.BlockSpec((BR, D), lambda p, i: (i, 0)),
            pl.BlockSpec((BR, 2), lambda p, i: (i, 0)),
            pl.BlockSpec((1, D), lambda p, i: (0, 0)),
            pl.BlockSpec((1, D), lambda p, i: (0, 0)),
        ],
        out_specs=pl.BlockSpec((BR, D), lambda p, i: (i, 0)),
        out_shape=jax.ShapeDtypeStruct((N, D), jnp.float32),
        scratch_shapes=[pltpu.VMEM((8, D), jnp.float32)],
    )(a0, a1, y, deg_t, gamma, beta)


def kernel(x, W, b, gamma, beta, edge_index):
    del b
    ei = edge_index.astype(jnp.int32)
    src3 = ei[0].reshape(NW, NG, CPG, CHUNK)
    dst3 = ei[1].reshape(NW, NG, CPG, CHUNK)

    zeros_np = jnp.zeros((NP,), jnp.float32)
    ones_c = jnp.ones((CHUNK,), jnp.float32)
    zeros_nd = jnp.zeros((NPR, D), jnp.float32)

    degf = _deg_sc(dst3, zeros_np, ones_c)
    deg_t = degf.reshape(NC, NP).T[:N]
    y = _matmul_tc(x, W, deg_t)
    acc = _scatter_sc(src3, dst3, y, zeros_nd)
    a0, a1 = acc[0, :N], acc[1, :N]
    out = _bn_tc(a0, a1, y, deg_t,
                 gamma.reshape(1, D), beta.reshape(1, D))
    return out


# speedup vs baseline: 38.5859x; 1.0011x over previous
import functools

import jax
import jax.numpy as jnp
from jax import lax
from jax.experimental import pallas as pl
from jax.experimental.pallas import tpu as pltpu
from jax.experimental.pallas import tpu_sc as plsc

N = 10000
E = 320000
D = 128

NC = 2
NS = 16
NW = NC * NS
EW = E // NW
CHUNK = 80
NCHUNK = EW // CHUNK
NP = 10240
NPR = 10112
ROWS_W = NPR // NS
DEGS_W = NP // NS
NG = 5
CPG = NCHUNK // NG

@functools.cache
def _deg_sc_build():
    return pl.kernel(
        _deg_sc_body,
        out_type=jax.ShapeDtypeStruct((NC * NP,), jnp.float32),
        mesh=plsc.VectorSubcoreMesh(core_axis_name="c", subcore_axis_name="s"),
        scratch_types=[
            pltpu.VMEM((CPG, CHUNK), jnp.int32),
            pltpu.VMEM((CHUNK,), jnp.float32),
            pltpu.VMEM_SHARED((NP,), jnp.float32),
            pltpu.SemaphoreType.DMA,
        ],
    )


def _deg_sc(dst3, zeros_np, ones_c):
    return _deg_sc_build()(dst3, zeros_np, ones_c)


def _deg_sc_body(dst_hbm, zeros_hbm, ones_hbm, out_hbm, idx_v, ones_v,
                 deg_sh, ssem):
    cid = lax.axis_index("c")
    sid = lax.axis_index("s")
    w = cid * NS + sid

    pltpu.sync_copy(zeros_hbm.at[pl.ds(sid * DEGS_W, DEGS_W)],
                    deg_sh.at[pl.ds(sid * DEGS_W, DEGS_W)])
    pltpu.sync_copy(ones_hbm, ones_v)
    plsc.subcore_barrier()

    def group(g, carry):
        pltpu.sync_copy(dst_hbm.at[w, g], idx_v)

        def body(j, c2):
            pltpu.async_copy(ones_v, deg_sh.at[idx_v.at[j]], ssem, add=True)
            return c2

        lax.fori_loop(0, CPG, body, 0)

        def drain(j, c2):
            pltpu.make_async_copy(ones_v, deg_sh.at[idx_v.at[0]], ssem).wait()
            return c2

        lax.fori_loop(0, CPG, drain, 0)
        return carry

    lax.fori_loop(0, NG, group, 0)
    plsc.subcore_barrier()

    pltpu.sync_copy(deg_sh.at[pl.ds(sid * DEGS_W, DEGS_W)],
                    out_hbm.at[pl.ds(cid * NP + sid * DEGS_W, DEGS_W)])


@functools.cache
def _scatter_sc_build():
    return pl.kernel(
        _scatter_sc_body,
        out_type=jax.ShapeDtypeStruct((NC, NPR, D), jnp.float32),
        mesh=plsc.VectorSubcoreMesh(core_axis_name="c", subcore_axis_name="s"),
        scratch_types=[
            pltpu.VMEM((CPG, CHUNK), jnp.int32),
            pltpu.VMEM((CPG, CHUNK), jnp.int32),
            pltpu.VMEM((CHUNK, D), jnp.float32),
            pltpu.VMEM((CHUNK, D), jnp.float32),
            pltpu.VMEM((CHUNK, D), jnp.float32),
            pltpu.VMEM((CHUNK, D), jnp.float32),
            pltpu.VMEM_SHARED((NPR, D), jnp.float32),
            pltpu.SemaphoreType.DMA,
            pltpu.SemaphoreType.DMA,
            pltpu.SemaphoreType.DMA,
            pltpu.SemaphoreType.DMA,
            pltpu.SemaphoreType.DMA,
            pltpu.SemaphoreType.DMA,
            pltpu.SemaphoreType.DMA,
            pltpu.SemaphoreType.DMA,
        ],
    )


def _scatter_sc(src3, dst3, y, zeros_nd):
    return _scatter_sc_build()(src3, dst3, y, zeros_nd)


def _scatter_sc_body(src_hbm, dst_hbm, y_hbm, zeros_hbm, out_hbm,
                     src_v, dst_v, rows0, rows1, rows2, rows3, acc_sh,
                     gsem0, gsem1, gsem2, gsem3,
                     ssem0, ssem1, ssem2, ssem3):
    cid = lax.axis_index("c")
    sid = lax.axis_index("s")
    w = cid * NS + sid
    rows = (rows0, rows1, rows2, rows3)
    gsem = (gsem0, gsem1, gsem2, gsem3)
    ssem = (ssem0, ssem1, ssem2, ssem3)

    pltpu.sync_copy(zeros_hbm.at[pl.ds(sid * ROWS_W, ROWS_W)],
                    acc_sh.at[pl.ds(sid * ROWS_W, ROWS_W)])
    plsc.subcore_barrier()

    def gather(j, b):
        return pltpu.async_copy(y_hbm.at[src_v.at[j]], rows[b], gsem[b])

    def scatter(j, b):
        return pltpu.async_copy(rows[b], acc_sh.at[dst_v.at[j]], ssem[b],
                                add=True)

    def wait_g(b):
        pltpu.make_async_copy(y_hbm.at[src_v.at[0]], rows[b], gsem[b]).wait()

    def wait_s(b):
        pltpu.make_async_copy(rows[b], acc_sh.at[dst_v.at[0]], ssem[b]).wait()

    def group(g, carry):
        pltpu.sync_copy(src_hbm.at[w, g], src_v)
        pltpu.sync_copy(dst_hbm.at[w, g], dst_v)
        gather(0, 0)
        gather(1, 1)
        gather(2, 2)
        wait_g(0); scatter(0, 0); gather(3, 3)
        wait_g(1); scatter(1, 1); wait_s(0); gather(4, 0)
        wait_g(2); scatter(2, 2); wait_s(1); gather(5, 1)
        wait_g(3); scatter(3, 3); wait_s(2); gather(6, 2)

        def body(i, c2):
            for p in range(4):
                j = 4 * i + p
                bn = (p + 3) % 4
                wait_g(p)
                scatter(j, p)
                wait_s(bn)
                gather(j + 3, bn)
            return c2

        lax.fori_loop(1, 5, body, 0)
        wait_g(0); scatter(20, 0); wait_s(3); gather(23, 3)
        wait_g(1); scatter(21, 1); wait_s(0); gather(24, 0)
        wait_g(2); scatter(22, 2); wait_s(1)
        wait_g(3); scatter(23, 3); wait_s(2)
        wait_g(0); scatter(24, 0); wait_s(3); wait_s(0)
        return carry

    lax.fori_loop(0, NG, group, 0)
    plsc.subcore_barrier()

    pltpu.sync_copy(acc_sh.at[pl.ds(sid * ROWS_W, ROWS_W)],
                    out_hbm.at[cid, pl.ds(sid * ROWS_W, ROWS_W)])


BR = 2000
NBLK = N // BR


def _dinv(dt_ref):
    return lax.rsqrt(1.0 + dt_ref[:, 0] + dt_ref[:, 1])


def _mm_body(x_ref, w_ref, dt_ref, o_ref):
    xw = jnp.dot(x_ref[...], w_ref[...], preferred_element_type=jnp.float32)
    o_ref[...] = xw * _dinv(dt_ref)[:, None]


def _matmul_tc(x, W, deg_t):
    return pl.pallas_call(
        _mm_body,
        grid=(NBLK,),
        in_specs=[
            pl.BlockSpec((BR, D), lambda i: (i, 0)),
            pl.BlockSpec((D, D), lambda i: (0, 0)),
            pl.BlockSpec((BR, 2), lambda i: (i, 0)),
        ],
        out_specs=pl.BlockSpec((BR, D), lambda i: (i, 0)),
        out_shape=jax.ShapeDtypeStruct((N, D), jnp.float32),
    )(x, W, deg_t)


def _bn_body(a0_ref, a1_ref, y_ref, dt_ref, g_ref, b_ref, o_ref, st_ref):
    p = pl.program_id(0)
    i = pl.program_id(1)
    z = (a0_ref[...] + a1_ref[...] + y_ref[...]) * _dinv(dt_ref)[:, None]

    @pl.when(p == 0)
    def _():
        @pl.when(i == 0)
        def _():
            st_ref[...] = jnp.zeros_like(st_ref)

        st_ref[0:1, :] += jnp.sum(z, axis=0, keepdims=True)
        st_ref[1:2, :] += jnp.sum(z * z, axis=0, keepdims=True)

        @pl.when(i == NBLK - 1)
        def _():
            mean = st_ref[0:1, :] * (1.0 / N)
            var = st_ref[1:2, :] * (1.0 / N) - mean * mean
            scale = lax.rsqrt(var + 1e-5) * g_ref[...]
            st_ref[2:3, :] = scale
            st_ref[3:4, :] = b_ref[...] - mean * scale

    @pl.when(p == 1)
    def _():
        o_ref[...] = jnp.maximum(z * st_ref[2:3, :] + st_ref[3:4, :], 0.0)


def _bn_tc(a0, a1, y, deg_t, gamma, beta):
    return pl.pallas_call(
        _bn_body,
        grid=(2, NBLK),
        in_specs=[
            pl.BlockSpec((BR, D), lambda p, i: (i, 0)),
            pl.BlockSpec((BR, D), lambda p, i: (i, 0)),
            pl.BlockSpec((BR, D), lambda p, i: (i, 0)),
            pl.BlockSpec((BR, 2), lambda p, i: (i, 0)),
            pl.BlockSpec((1, D), lambda p, i: (0, 0)),
            pl.BlockSpec((1, D), lambda p, i: (0, 0)),
        ],
        out_specs=pl.BlockSpec((BR, D), lambda p, i: (i, 0)),
        out_shape=jax.ShapeDtypeStruct((N, D), jnp.float32),
        scratch_shapes=[pltpu.VMEM((8, D), jnp.float32)],
    )(a0, a1, y, deg_t, gamma, beta)


def kernel(x, W, b, gamma, beta, edge_index):
    del b
    ei = edge_index.astype(jnp.int32)
    src3 = ei[0].reshape(NW, NG, CPG, CHUNK)
    dst3 = ei[1].reshape(NW, NG, CPG, CHUNK)

    zeros_np = jnp.zeros((NP,), jnp.float32)
    ones_c = jnp.ones((CHUNK,), jnp.float32)
    zeros_nd = jnp.zeros((NPR, D), jnp.float32)

    degf = _deg_sc(dst3, zeros_np, ones_c)
    deg_t = degf.reshape(NC, NP).T[:N]
    y = _matmul_tc(x, W, deg_t)
    acc = _scatter_sc(src3, dst3, y, zeros_nd)
    a0, a1 = acc[0, :N], acc[1, :N]
    out = _bn_tc(a0, a1, y, deg_t,
                 gamma.reshape(1, D), beta.reshape(1, D))
    return out


# speedup vs baseline: 41.8184x; 1.0838x over previous
import functools

import jax
import jax.numpy as jnp
from jax import lax
from jax.experimental import pallas as pl
from jax.experimental.pallas import tpu as pltpu
from jax.experimental.pallas import tpu_sc as plsc

N = 10000
E = 320000
D = 128

NC = 2
NS = 16
NW = NC * NS
EW = E // NW
CHUNK = 80
NCHUNK = EW // CHUNK
NP = 10240
NPR = 10112
ROWS_W = NPR // NS
DEGS_W = NP // NS
NG = 5
CPG = NCHUNK // NG
LAST_W = N - (NS - 1) * (NPR // NS)

@functools.cache
def _deg_sc_build():
    return pl.kernel(
        _deg_sc_body,
        out_type=jax.ShapeDtypeStruct((NC * NP,), jnp.float32),
        mesh=plsc.VectorSubcoreMesh(core_axis_name="c", subcore_axis_name="s"),
        scratch_types=[
            pltpu.VMEM((CPG, CHUNK), jnp.int32),
            pltpu.VMEM((CHUNK,), jnp.float32),
            pltpu.VMEM_SHARED((NP,), jnp.float32),
            pltpu.SemaphoreType.DMA,
        ],
    )


def _deg_sc(dst3, zeros_np, ones_c):
    return _deg_sc_build()(dst3, zeros_np, ones_c)


def _deg_sc_body(dst_hbm, zeros_hbm, ones_hbm, out_hbm, idx_v, ones_v,
                 deg_sh, ssem):
    cid = lax.axis_index("c")
    sid = lax.axis_index("s")
    w = cid * NS + sid

    pltpu.sync_copy(zeros_hbm.at[pl.ds(sid * DEGS_W, DEGS_W)],
                    deg_sh.at[pl.ds(sid * DEGS_W, DEGS_W)])
    pltpu.sync_copy(ones_hbm, ones_v)
    plsc.subcore_barrier()

    def group(g, carry):
        pltpu.sync_copy(dst_hbm.at[w, g], idx_v)

        def body(j, c2):
            pltpu.async_copy(ones_v, deg_sh.at[idx_v.at[j]], ssem, add=True)
            return c2

        lax.fori_loop(0, CPG, body, 0)

        def drain(j, c2):
            pltpu.make_async_copy(ones_v, deg_sh.at[idx_v.at[0]], ssem).wait()
            return c2

        lax.fori_loop(0, CPG, drain, 0)
        return carry

    lax.fori_loop(0, NG, group, 0)
    plsc.subcore_barrier()

    pltpu.sync_copy(deg_sh.at[pl.ds(sid * DEGS_W, DEGS_W)],
                    out_hbm.at[pl.ds(cid * NP + sid * DEGS_W, DEGS_W)])


@functools.cache
def _scatter_sc_build():
    return pl.kernel(
        _scatter_sc_body,
        out_type=jax.ShapeDtypeStruct((NC, N, D), jnp.float32),
        mesh=plsc.VectorSubcoreMesh(core_axis_name="c", subcore_axis_name="s"),
        scratch_types=[
            pltpu.VMEM((CPG, CHUNK), jnp.int32),
            pltpu.VMEM((CPG, CHUNK), jnp.int32),
            pltpu.VMEM((CHUNK, D), jnp.float32),
            pltpu.VMEM((CHUNK, D), jnp.float32),
            pltpu.VMEM((CHUNK, D), jnp.float32),
            pltpu.VMEM((CHUNK, D), jnp.float32),
            pltpu.VMEM_SHARED((NPR, D), jnp.float32),
            pltpu.SemaphoreType.DMA,
            pltpu.SemaphoreType.DMA,
            pltpu.SemaphoreType.DMA,
            pltpu.SemaphoreType.DMA,
            pltpu.SemaphoreType.DMA,
            pltpu.SemaphoreType.DMA,
            pltpu.SemaphoreType.DMA,
            pltpu.SemaphoreType.DMA,
        ],
    )


def _scatter_sc(src3, dst3, y, zeros_nd):
    return _scatter_sc_build()(src3, dst3, y, zeros_nd)


def _scatter_sc_body(src_hbm, dst_hbm, y_hbm, zeros_hbm, out_hbm,
                     src_v, dst_v, rows0, rows1, rows2, rows3, acc_sh,
                     gsem0, gsem1, gsem2, gsem3,
                     ssem0, ssem1, ssem2, ssem3):
    cid = lax.axis_index("c")
    sid = lax.axis_index("s")
    w = cid * NS + sid
    rows = (rows0, rows1, rows2, rows3)
    gsem = (gsem0, gsem1, gsem2, gsem3)
    ssem = (ssem0, ssem1, ssem2, ssem3)

    @pl.when(sid < NS - 1)
    def _():
        pltpu.sync_copy(zeros_hbm, acc_sh.at[pl.ds(sid * ROWS_W, ROWS_W)])

    @pl.when(sid == NS - 1)
    def _():
        pltpu.sync_copy(zeros_hbm.at[pl.ds(0, LAST_W)],
                        acc_sh.at[pl.ds((NS - 1) * ROWS_W, LAST_W)])

    plsc.subcore_barrier()

    def gather(j, b):
        return pltpu.async_copy(y_hbm.at[src_v.at[j]], rows[b], gsem[b])

    def scatter(j, b):
        return pltpu.async_copy(rows[b], acc_sh.at[dst_v.at[j]], ssem[b],
                                add=True)

    def wait_g(b):
        pltpu.make_async_copy(y_hbm.at[src_v.at[0]], rows[b], gsem[b]).wait()

    def wait_s(b):
        pltpu.make_async_copy(rows[b], acc_sh.at[dst_v.at[0]], ssem[b]).wait()

    def group(g, carry):
        pltpu.sync_copy(src_hbm.at[w, g], src_v)
        pltpu.sync_copy(dst_hbm.at[w, g], dst_v)
        gather(0, 0)
        gather(1, 1)
        gather(2, 2)
        wait_g(0); scatter(0, 0); gather(3, 3)
        wait_g(1); scatter(1, 1); wait_s(0); gather(4, 0)
        wait_g(2); scatter(2, 2); wait_s(1); gather(5, 1)
        wait_g(3); scatter(3, 3); wait_s(2); gather(6, 2)

        def body(i, c2):
            for p in range(4):
                j = 4 * i + p
                bn = (p + 3) % 4
                wait_g(p)
                scatter(j, p)
                wait_s(bn)
                gather(j + 3, bn)
            return c2

        lax.fori_loop(1, 5, body, 0)
        wait_g(0); scatter(20, 0); wait_s(3); gather(23, 3)
        wait_g(1); scatter(21, 1); wait_s(0); gather(24, 0)
        wait_g(2); scatter(22, 2); wait_s(1)
        wait_g(3); scatter(23, 3); wait_s(2)
        wait_g(0); scatter(24, 0); wait_s(3); wait_s(0)
        return carry

    lax.fori_loop(0, NG, group, 0)
    plsc.subcore_barrier()

    @pl.when(sid < NS - 1)
    def _():
        pltpu.sync_copy(acc_sh.at[pl.ds(sid * ROWS_W, ROWS_W)],
                        out_hbm.at[cid, pl.ds(sid * ROWS_W, ROWS_W)])

    @pl.when(sid == NS - 1)
    def _():
        pltpu.sync_copy(acc_sh.at[pl.ds((NS - 1) * ROWS_W, LAST_W)],
                        out_hbm.at[cid, pl.ds((NS - 1) * ROWS_W, LAST_W)])


def _dinv(dt_ref):
    return lax.rsqrt(1.0 + dt_ref[:, 0] + dt_ref[:, 1])


def _mm_body(x_ref, w_ref, dt_ref, o_ref):
    xw = jnp.dot(x_ref[...], w_ref[...], preferred_element_type=jnp.float32)
    o_ref[...] = xw * _dinv(dt_ref)[:, None]


def _matmul_tc(x, W, deg_t):
    return pl.pallas_call(
        _mm_body,
        out_shape=jax.ShapeDtypeStruct((N, D), jnp.float32),
    )(x, W, deg_t)


def _bn_body(acc_ref, y_ref, dt_ref, g_ref, b_ref, o_ref):
    z = (acc_ref[0] + acc_ref[1] + y_ref[...]) * _dinv(dt_ref)[:, None]
    mean = jnp.sum(z, axis=0, keepdims=True) * (1.0 / N)
    var = jnp.sum(z * z, axis=0, keepdims=True) * (1.0 / N) - mean * mean
    scale = lax.rsqrt(var + 1e-5) * g_ref[...]
    shift = b_ref[...] - mean * scale
    o_ref[...] = jnp.maximum(z * scale + shift, 0.0)


def _bn_tc(acc, y, deg_t, gamma, beta):
    return pl.pallas_call(
        _bn_body,
        out_shape=jax.ShapeDtypeStruct((N, D), jnp.float32),
    )(acc, y, deg_t, gamma, beta)


def kernel(x, W, b, gamma, beta, edge_index):
    del b
    ei = edge_index.astype(jnp.int32)
    src3 = ei[0].reshape(NW, NG, CPG, CHUNK)
    dst3 = ei[1].reshape(NW, NG, CPG, CHUNK)

    zeros_np = jnp.zeros((NP,), jnp.float32)
    ones_c = jnp.ones((CHUNK,), jnp.float32)
    zeros_rw = jnp.zeros((ROWS_W, D), jnp.float32)

    degf = _deg_sc(dst3, zeros_np, ones_c)
    deg_t = degf.reshape(NC, NP).T[:N]
    y = _matmul_tc(x, W, deg_t)
    acc = _scatter_sc(src3, dst3, y, zeros_rw)
    out = _bn_tc(acc, y, deg_t,
                 gamma.reshape(1, D), beta.reshape(1, D))
    return out


# speedup vs baseline: 43.1121x; 1.0309x over previous
import functools

import jax
import jax.numpy as jnp
from jax import lax
from jax.experimental import pallas as pl
from jax.experimental.pallas import tpu as pltpu
from jax.experimental.pallas import tpu_sc as plsc

N = 10000
E = 320000
D = 128

NC = 2
NS = 16
NW = NC * NS
EW = E // NW
CHUNK = 100
NCHUNK = EW // CHUNK
NP = 10240
NPR = 10112
ROWS_W = NPR // NS
DEGS_W = NP // NS
NG = 4
CPG = NCHUNK // NG
LAST_W = N - (NS - 1) * (NPR // NS)

@functools.cache
def _deg_sc_build():
    return pl.kernel(
        _deg_sc_body,
        out_type=jax.ShapeDtypeStruct((NC * NP,), jnp.float32),
        mesh=plsc.VectorSubcoreMesh(core_axis_name="c", subcore_axis_name="s"),
        scratch_types=[
            pltpu.VMEM((CPG, CHUNK), jnp.int32),
            pltpu.VMEM((CHUNK,), jnp.float32),
            pltpu.VMEM_SHARED((NP,), jnp.float32),
            pltpu.SemaphoreType.DMA,
        ],
    )


def _deg_sc(dst3, zeros_np, ones_c):
    return _deg_sc_build()(dst3, zeros_np, ones_c)


def _deg_sc_body(dst_hbm, zeros_hbm, ones_hbm, out_hbm, idx_v, ones_v,
                 deg_sh, ssem):
    cid = lax.axis_index("c")
    sid = lax.axis_index("s")
    w = cid * NS + sid

    pltpu.sync_copy(zeros_hbm.at[pl.ds(sid * DEGS_W, DEGS_W)],
                    deg_sh.at[pl.ds(sid * DEGS_W, DEGS_W)])
    pltpu.sync_copy(ones_hbm, ones_v)
    plsc.subcore_barrier()

    def group(g, carry):
        pltpu.sync_copy(dst_hbm.at[w, g], idx_v)

        def body(j, c2):
            pltpu.async_copy(ones_v, deg_sh.at[idx_v.at[j]], ssem, add=True)
            return c2

        lax.fori_loop(0, CPG, body, 0)

        def drain(j, c2):
            pltpu.make_async_copy(ones_v, deg_sh.at[idx_v.at[0]], ssem).wait()
            return c2

        lax.fori_loop(0, CPG, drain, 0)
        return carry

    lax.fori_loop(0, NG, group, 0)
    plsc.subcore_barrier()

    pltpu.sync_copy(deg_sh.at[pl.ds(sid * DEGS_W, DEGS_W)],
                    out_hbm.at[pl.ds(cid * NP + sid * DEGS_W, DEGS_W)])


@functools.cache
def _scatter_sc_build():
    return pl.kernel(
        _scatter_sc_body,
        out_type=jax.ShapeDtypeStruct((NC, N, D), jnp.float32),
        mesh=plsc.VectorSubcoreMesh(core_axis_name="c", subcore_axis_name="s"),
        scratch_types=[
            pltpu.VMEM((CPG, CHUNK), jnp.int32),
            pltpu.VMEM((CPG, CHUNK), jnp.int32),
            pltpu.VMEM((CHUNK, D), jnp.float32),
            pltpu.VMEM((CHUNK, D), jnp.float32),
            pltpu.VMEM((CHUNK, D), jnp.float32),
            pltpu.VMEM_SHARED((NPR, D), jnp.float32),
            pltpu.SemaphoreType.DMA,
            pltpu.SemaphoreType.DMA,
            pltpu.SemaphoreType.DMA,
            pltpu.SemaphoreType.DMA,
            pltpu.SemaphoreType.DMA,
            pltpu.SemaphoreType.DMA,
        ],
    )


def _scatter_sc(src3, dst3, y, zeros_nd):
    return _scatter_sc_build()(src3, dst3, y, zeros_nd)


def _scatter_sc_body(src_hbm, dst_hbm, y_hbm, zeros_hbm, out_hbm,
                     src_v, dst_v, rows0, rows1, rows2, acc_sh,
                     gsem0, gsem1, gsem2, ssem0, ssem1, ssem2):
    cid = lax.axis_index("c")
    sid = lax.axis_index("s")
    w = cid * NS + sid
    rows = (rows0, rows1, rows2)
    gsem = (gsem0, gsem1, gsem2)
    ssem = (ssem0, ssem1, ssem2)

    @pl.when(sid < NS - 1)
    def _():
        pltpu.sync_copy(zeros_hbm, acc_sh.at[pl.ds(sid * ROWS_W, ROWS_W)])

    @pl.when(sid == NS - 1)
    def _():
        pltpu.sync_copy(zeros_hbm.at[pl.ds(0, LAST_W)],
                        acc_sh.at[pl.ds((NS - 1) * ROWS_W, LAST_W)])

    plsc.subcore_barrier()

    def gather(j, b):
        return pltpu.async_copy(y_hbm.at[src_v.at[j]], rows[b], gsem[b])

    def scatter(j, b):
        return pltpu.async_copy(rows[b], acc_sh.at[dst_v.at[j]], ssem[b],
                                add=True)

    def wait_g(b):
        pltpu.make_async_copy(y_hbm.at[src_v.at[0]], rows[b], gsem[b]).wait()

    def wait_s(b):
        pltpu.make_async_copy(rows[b], acc_sh.at[dst_v.at[0]], ssem[b]).wait()

    def group(g, carry):
        pltpu.sync_copy(src_hbm.at[w, g], src_v)
        pltpu.sync_copy(dst_hbm.at[w, g], dst_v)
        gather(0, 0)
        gather(1, 1)
        wait_g(0); scatter(0, 0); gather(2, 2)

        def body(i, c2):
            for p in range(3):
                j = 3 * i + p + 1
                b = (p + 1) % 3
                bn = p
                wait_g(b)
                scatter(j, b)
                wait_s(bn)
                gather(j + 2, bn)
            return c2

        lax.fori_loop(0, 7, body, 0)
        wait_g(1); scatter(22, 1); wait_s(0); gather(24, 0)
        wait_g(2); scatter(23, 2); wait_s(1)
        wait_g(0); scatter(24, 0); wait_s(2); wait_s(0)
        return carry

    lax.fori_loop(0, NG, group, 0)
    plsc.subcore_barrier()

    @pl.when(sid < NS - 1)
    def _():
        pltpu.sync_copy(acc_sh.at[pl.ds(sid * ROWS_W, ROWS_W)],
                        out_hbm.at[cid, pl.ds(sid * ROWS_W, ROWS_W)])

    @pl.when(sid == NS - 1)
    def _():
        pltpu.sync_copy(acc_sh.at[pl.ds((NS - 1) * ROWS_W, LAST_W)],
                        out_hbm.at[cid, pl.ds((NS - 1) * ROWS_W, LAST_W)])


def _dinv(dt_ref):
    return lax.rsqrt(1.0 + dt_ref[:, 0] + dt_ref[:, 1])


def _mm_body(x_ref, w_ref, dt_ref, o_ref):
    xw = jnp.dot(x_ref[...], w_ref[...], preferred_element_type=jnp.float32)
    o_ref[...] = xw * _dinv(dt_ref)[:, None]


def _matmul_tc(x, W, deg_t):
    return pl.pallas_call(
        _mm_body,
        out_shape=jax.ShapeDtypeStruct((N, D), jnp.float32),
    )(x, W, deg_t)


def _bn_body(acc_ref, y_ref, dt_ref, g_ref, b_ref, o_ref):
    z = (acc_ref[0] + acc_ref[1] + y_ref[...]) * _dinv(dt_ref)[:, None]
    mean = jnp.sum(z, axis=0, keepdims=True) * (1.0 / N)
    var = jnp.sum(z * z, axis=0, keepdims=True) * (1.0 / N) - mean * mean
    scale = lax.rsqrt(var + 1e-5) * g_ref[...]
    shift = b_ref[...] - mean * scale
    o_ref[...] = jnp.maximum(z * scale + shift, 0.0)


def _bn_tc(acc, y, deg_t, gamma, beta):
    return pl.pallas_call(
        _bn_body,
        out_shape=jax.ShapeDtypeStruct((N, D), jnp.float32),
    )(acc, y, deg_t, gamma, beta)


def kernel(x, W, b, gamma, beta, edge_index):
    del b
    ei = edge_index.astype(jnp.int32)
    src3 = ei[0].reshape(NW, NG, CPG, CHUNK)
    dst3 = ei[1].reshape(NW, NG, CPG, CHUNK)

    zeros_np = jnp.zeros((NP,), jnp.float32)
    ones_c = jnp.ones((CHUNK,), jnp.float32)
    zeros_rw = jnp.zeros((ROWS_W, D), jnp.float32)

    degf = _deg_sc(dst3, zeros_np, ones_c)
    deg_t = degf.reshape(NC, NP).T[:N]
    y = _matmul_tc(x, W, deg_t)
    acc = _scatter_sc(src3, dst3, y, zeros_rw)
    out = _bn_tc(acc, y, deg_t,
                 gamma.reshape(1, D), beta.reshape(1, D))
    return out
